# initial kernel scaffold (unmeasured)
import jax
import jax.numpy as jnp
from jax import lax
from jax.experimental import pallas as pl
from jax.experimental.pallas import tpu as pltpu

N_DEV = 4
HPD = 8
SQ = 2048
SKV = 2048
DH = 128
DIN = 1024
DSH = HPD * DH
BQ = 512
SCALE = 0.08838834764831843
WINDOW = 128
NGLOB = 32
CH = SQ // N_DEV


def _attn_body(x_ref, wq_ref, k_ref, v_ref, ctx_ref):
    qb = pl.program_id(1)
    q = jnp.dot(x_ref[...], wq_ref[...], preferred_element_type=jnp.float32)
    k = k_ref[:, 0, :]
    s = lax.dot_general(
        q, k, (((1,), (1,)), ((), ())), preferred_element_type=jnp.float32
    ) * SCALE
    qi = qb * BQ + lax.broadcasted_iota(jnp.int32, (BQ, SKV), 0)
    ki = lax.broadcasted_iota(jnp.int32, (BQ, SKV), 1)
    mask = (jnp.abs(qi - ki) <= WINDOW) | (ki < NGLOB) | (qi < NGLOB)
    s = jnp.where(mask, s, -1e9)
    m = jnp.max(s, axis=1, keepdims=True)
    w = jnp.exp(s - m)
    w = w / jnp.sum(w, axis=1, keepdims=True)
    ctx_ref[...] = jnp.dot(w, v_ref[:, 0, :], preferred_element_type=jnp.float32)


def _attention(x2, Wq, K, V):
    grid = (HPD, SQ // BQ)
    return pl.pallas_call(
        _attn_body,
        grid=grid,
        in_specs=[
            pl.BlockSpec((BQ, DIN), lambda h, qb: (qb, 0)),
            pl.BlockSpec((DIN, DH), lambda h, qb: (0, h)),
            pl.BlockSpec((SKV, 1, DH), lambda h, qb: (0, h, 0)),
            pl.BlockSpec((SKV, 1, DH), lambda h, qb: (0, h, 0)),
        ],
        out_specs=pl.BlockSpec((BQ, DH), lambda h, qb: (qb, h)),
        out_shape=jax.ShapeDtypeStruct((SQ, DSH), jnp.float32),
        compiler_params=pltpu.CompilerParams(
            dimension_semantics=("arbitrary", "arbitrary"),
        ),
    )(x2, Wq, K, V)


def _ar_body(ctx_ref, wo_ref, out_ref, comm_ref,
             rs_send, rs_recv, ag_send, ag_recv):
    i = lax.axis_index("i")
    left = lax.rem(i + N_DEV - 1, N_DEV)
    right = lax.rem(i + 1, N_DEV)

    barrier = pltpu.get_barrier_semaphore()
    for nbr in (left, right):
        pl.semaphore_signal(
            barrier, inc=1, device_id=(nbr,),
            device_id_type=pl.DeviceIdType.MESH,
        )
    pl.semaphore_wait(barrier, 2)

    out_ref[...] = jnp.dot(
        ctx_ref[...], wo_ref[...], preferred_element_type=jnp.float32
    )

    for s in range(N_DEV - 1):
        send_c = lax.rem(i - s + N_DEV, N_DEV)
        recv_c = lax.rem(i - s - 1 + N_DEV, N_DEV)
        rdma = pltpu.make_async_remote_copy(
            src_ref=out_ref.at[pl.ds(send_c * CH, CH), :],
            dst_ref=comm_ref.at[s],
            send_sem=rs_send.at[s],
            recv_sem=rs_recv.at[s],
            device_id=(right,),
            device_id_type=pl.DeviceIdType.MESH,
        )
        rdma.start()
        rdma.wait()
        acc = out_ref[pl.ds(recv_c * CH, CH), :] + comm_ref[s]
        out_ref[pl.ds(recv_c * CH, CH), :] = acc

    for s in range(N_DEV - 1):
        c = lax.rem(i + 1 - s + N_DEV, N_DEV)
        rdma = pltpu.make_async_remote_copy(
            src_ref=out_ref.at[pl.ds(c * CH, CH), :],
            dst_ref=out_ref.at[pl.ds(c * CH, CH), :],
            send_sem=ag_send.at[s],
            recv_sem=ag_recv.at[s],
            device_id=(right,),
            device_id_type=pl.DeviceIdType.MESH,
        )
        rdma.start()
        rdma.wait()


def _outproj_allreduce(ctx, Wo):
    return pl.pallas_call(
        _ar_body,
        in_specs=[
            pl.BlockSpec(memory_space=pltpu.VMEM),
            pl.BlockSpec(memory_space=pltpu.VMEM),
        ],
        out_specs=pl.BlockSpec(memory_space=pltpu.VMEM),
        out_shape=jax.ShapeDtypeStruct((SQ, DIN), jnp.float32),
        scratch_shapes=[
            pltpu.VMEM((N_DEV - 1, CH, DIN), jnp.float32),
            pltpu.SemaphoreType.DMA((N_DEV - 1,)),
            pltpu.SemaphoreType.DMA((N_DEV - 1,)),
            pltpu.SemaphoreType.DMA((N_DEV - 1,)),
            pltpu.SemaphoreType.DMA((N_DEV - 1,)),
        ],
        compiler_params=pltpu.CompilerParams(collective_id=0),
    )(ctx, Wo)


def kernel(x, Wq, K_ext, V_ext, Wo):
    i = lax.axis_index("i")
    h0 = i * HPD
    x2 = x.reshape(SQ, DIN)
    K = lax.dynamic_slice(K_ext, (0, 0, h0, 0), (1, SKV, HPD, DH)).reshape(
        SKV, HPD, DH
    )
    V = lax.dynamic_slice(V_ext, (0, 0, h0, 0), (1, SKV, HPD, DH)).reshape(
        SKV, HPD, DH
    )
    ctx = _attention(x2, Wq, K, V)
    out = _outproj_allreduce(ctx, Wo)
    return out.reshape(1, SQ, DIN)


# baseline (device time: 300767 ns/iter reference)
import jax
import jax.numpy as jnp
from jax import lax
from jax.experimental import pallas as pl
from jax.experimental.pallas import tpu as pltpu

N_DEV = 4
HPD = 8
SQ = 2048
SKV = 2048
DH = 128
DIN = 1024
DSH = HPD * DH
BQ = 512
SCALE = 0.08838834764831843
WINDOW = 128
NGLOB = 32
CH = SQ // N_DEV


def _attn_body(x_ref, wq_ref, k_ref, v_ref, ctx_ref):
    qb = pl.program_id(1)
    q = jnp.dot(x_ref[...], wq_ref[...], preferred_element_type=jnp.float32)
    k = k_ref[0]
    s = lax.dot_general(
        q, k, (((1,), (1,)), ((), ())), preferred_element_type=jnp.float32
    ) * SCALE
    qi = qb * BQ + lax.broadcasted_iota(jnp.int32, (BQ, SKV), 0)
    ki = lax.broadcasted_iota(jnp.int32, (BQ, SKV), 1)
    mask = (jnp.abs(qi - ki) <= WINDOW) | (ki < NGLOB) | (qi < NGLOB)
    s = jnp.where(mask, s, -1e9)
    m = jnp.max(s, axis=1, keepdims=True)
    w = jnp.exp(s - m)
    w = w / jnp.sum(w, axis=1, keepdims=True)
    ctx_ref[...] = jnp.dot(w, v_ref[0], preferred_element_type=jnp.float32)


def _attention(x2, Wq, K, V):
    grid = (HPD, SQ // BQ)
    return pl.pallas_call(
        _attn_body,
        grid=grid,
        in_specs=[
            pl.BlockSpec((BQ, DIN), lambda h, qb: (qb, 0)),
            pl.BlockSpec((DIN, DH), lambda h, qb: (0, h)),
            pl.BlockSpec((1, SKV, DH), lambda h, qb: (h, 0, 0)),
            pl.BlockSpec((1, SKV, DH), lambda h, qb: (h, 0, 0)),
        ],
        out_specs=pl.BlockSpec((BQ, DH), lambda h, qb: (qb, h)),
        out_shape=jax.ShapeDtypeStruct((SQ, DSH), jnp.float32),
        compiler_params=pltpu.CompilerParams(
            dimension_semantics=("arbitrary", "arbitrary"),
        ),
    )(x2, Wq, K, V)


def _ar_body(ctx_ref, wo_ref, out_ref, comm_ref,
             rs_send, rs_recv, ag_send, ag_recv):
    i = lax.axis_index("i")
    left = lax.rem(i + N_DEV - 1, N_DEV)
    right = lax.rem(i + 1, N_DEV)

    barrier = pltpu.get_barrier_semaphore()
    for nbr in (left, right):
        pl.semaphore_signal(
            barrier, inc=1, device_id=(nbr,),
            device_id_type=pl.DeviceIdType.MESH,
        )
    pl.semaphore_wait(barrier, 2)

    out_ref[...] = jnp.dot(
        ctx_ref[...], wo_ref[...], preferred_element_type=jnp.float32
    )

    for s in range(N_DEV - 1):
        send_c = lax.rem(i - s + N_DEV, N_DEV)
        recv_c = lax.rem(i - s - 1 + N_DEV, N_DEV)
        rdma = pltpu.make_async_remote_copy(
            src_ref=out_ref.at[pl.ds(send_c * CH, CH), :],
            dst_ref=comm_ref.at[s],
            send_sem=rs_send.at[s],
            recv_sem=rs_recv.at[s],
            device_id=(right,),
            device_id_type=pl.DeviceIdType.MESH,
        )
        rdma.start()
        rdma.wait()
        acc = out_ref[pl.ds(recv_c * CH, CH), :] + comm_ref[s]
        out_ref[pl.ds(recv_c * CH, CH), :] = acc

    for s in range(N_DEV - 1):
        c = lax.rem(i + 1 - s + N_DEV, N_DEV)
        rdma = pltpu.make_async_remote_copy(
            src_ref=out_ref.at[pl.ds(c * CH, CH), :],
            dst_ref=out_ref.at[pl.ds(c * CH, CH), :],
            send_sem=ag_send.at[s],
            recv_sem=ag_recv.at[s],
            device_id=(right,),
            device_id_type=pl.DeviceIdType.MESH,
        )
        rdma.start()
        rdma.wait()


def _outproj_allreduce(ctx, Wo):
    return pl.pallas_call(
        _ar_body,
        in_specs=[
            pl.BlockSpec(memory_space=pltpu.VMEM),
            pl.BlockSpec(memory_space=pltpu.VMEM),
        ],
        out_specs=pl.BlockSpec(memory_space=pltpu.VMEM),
        out_shape=jax.ShapeDtypeStruct((SQ, DIN), jnp.float32),
        scratch_shapes=[
            pltpu.VMEM((N_DEV - 1, CH, DIN), jnp.float32),
            pltpu.SemaphoreType.DMA((N_DEV - 1,)),
            pltpu.SemaphoreType.DMA((N_DEV - 1,)),
            pltpu.SemaphoreType.DMA((N_DEV - 1,)),
            pltpu.SemaphoreType.DMA((N_DEV - 1,)),
        ],
        compiler_params=pltpu.CompilerParams(collective_id=0),
    )(ctx, Wo)


def kernel(x, Wq, K_ext, V_ext, Wo):
    i = lax.axis_index("i")
    h0 = i * HPD
    x2 = x.reshape(SQ, DIN)
    K = lax.dynamic_slice(K_ext, (0, 0, h0, 0), (1, SKV, HPD, DH)).reshape(
        SKV, HPD, DH
    ).transpose(1, 0, 2)
    V = lax.dynamic_slice(V_ext, (0, 0, h0, 0), (1, SKV, HPD, DH)).reshape(
        SKV, HPD, DH
    ).transpose(1, 0, 2)
    ctx = _attention(x2, Wq, K, V)
    out = _outproj_allreduce(ctx, Wo)
    return out.reshape(1, SQ, DIN)


# device time: 233174 ns/iter; 1.2899x vs baseline; 1.2899x over previous
import jax
import jax.numpy as jnp
from jax import lax
from jax.experimental import pallas as pl
from jax.experimental.pallas import tpu as pltpu

N_DEV = 4
HPD = 8
SQ = 2048
SKV = 2048
DH = 128
DIN = 1024
DSH = HPD * DH
BQ = 512
SCALE = 0.08838834764831843
WINDOW = 128
NGLOB = 32
CH = SQ // N_DEV


def _attn_body(x_ref, wq_ref, k_ref, v_ref, ctx_ref):
    qb = pl.program_id(1)
    q = jnp.dot(x_ref[...], wq_ref[...], preferred_element_type=jnp.float32)
    k = k_ref[0]
    s = lax.dot_general(
        q, k, (((1,), (1,)), ((), ())), preferred_element_type=jnp.float32
    ) * SCALE
    qi = qb * BQ + lax.broadcasted_iota(jnp.int32, (BQ, SKV), 0)
    ki = lax.broadcasted_iota(jnp.int32, (BQ, SKV), 1)
    mask = (jnp.abs(qi - ki) <= WINDOW) | (ki < NGLOB) | (qi < NGLOB)
    s = jnp.where(mask, s, -1e9)
    m = jnp.max(s, axis=1, keepdims=True)
    w = jnp.exp(s - m)
    w = w / jnp.sum(w, axis=1, keepdims=True)
    ctx_ref[...] = jnp.dot(w, v_ref[0], preferred_element_type=jnp.float32)


def _attention(x2, Wq, K, V):
    grid = (HPD, SQ // BQ)
    return pl.pallas_call(
        _attn_body,
        grid=grid,
        in_specs=[
            pl.BlockSpec((BQ, DIN), lambda h, qb: (qb, 0)),
            pl.BlockSpec((DIN, DH), lambda h, qb: (0, h)),
            pl.BlockSpec((1, SKV, DH), lambda h, qb: (h, 0, 0)),
            pl.BlockSpec((1, SKV, DH), lambda h, qb: (h, 0, 0)),
        ],
        out_specs=pl.BlockSpec((BQ, DH), lambda h, qb: (qb, h)),
        out_shape=jax.ShapeDtypeStruct((SQ, DSH), jnp.float32),
        compiler_params=pltpu.CompilerParams(
            dimension_semantics=("arbitrary", "arbitrary"),
        ),
    )(x2, Wq, K, V)


HC = DIN // 2


def _ar_body(ctx_ref, wo_ref, out_ref, cw_comm, ccw_comm,
             cw_send, cw_recv, ccw_send, ccw_recv,
             agcw_send, agcw_recv, agccw_send, agccw_recv):
    i = lax.axis_index("i")
    left = lax.rem(i + N_DEV - 1, N_DEV)
    right = lax.rem(i + 1, N_DEV)

    barrier = pltpu.get_barrier_semaphore()
    for nbr in (left, right):
        pl.semaphore_signal(
            barrier, inc=1, device_id=(nbr,),
            device_id_type=pl.DeviceIdType.MESH,
        )
    pl.semaphore_wait(barrier, 2)

    out_ref[...] = jnp.dot(
        ctx_ref[...], wo_ref[...], preferred_element_type=jnp.float32
    )

    for s in range(N_DEV - 1):
        cw_sc = lax.rem(i - s + N_DEV, N_DEV)
        cw_rc = lax.rem(i - s - 1 + N_DEV, N_DEV)
        ccw_sc = lax.rem(i + s, N_DEV)
        ccw_rc = lax.rem(i + s + 1, N_DEV)
        cw = pltpu.make_async_remote_copy(
            src_ref=out_ref.at[pl.ds(cw_sc * CH, CH), pl.ds(0, HC)],
            dst_ref=cw_comm.at[s],
            send_sem=cw_send.at[s],
            recv_sem=cw_recv.at[s],
            device_id=(right,),
            device_id_type=pl.DeviceIdType.MESH,
        )
        ccw = pltpu.make_async_remote_copy(
            src_ref=out_ref.at[pl.ds(ccw_sc * CH, CH), pl.ds(HC, HC)],
            dst_ref=ccw_comm.at[s],
            send_sem=ccw_send.at[s],
            recv_sem=ccw_recv.at[s],
            device_id=(left,),
            device_id_type=pl.DeviceIdType.MESH,
        )
        cw.start()
        ccw.start()
        cw.wait()
        ccw.wait()
        acc = out_ref[pl.ds(cw_rc * CH, CH), pl.ds(0, HC)] + cw_comm[s]
        out_ref[pl.ds(cw_rc * CH, CH), pl.ds(0, HC)] = acc
        acc = out_ref[pl.ds(ccw_rc * CH, CH), pl.ds(HC, HC)] + ccw_comm[s]
        out_ref[pl.ds(ccw_rc * CH, CH), pl.ds(HC, HC)] = acc

    for s in range(N_DEV - 1):
        cw_c = lax.rem(i + 1 - s + N_DEV, N_DEV)
        ccw_c = lax.rem(i - 1 + s + N_DEV, N_DEV)
        cw = pltpu.make_async_remote_copy(
            src_ref=out_ref.at[pl.ds(cw_c * CH, CH), pl.ds(0, HC)],
            dst_ref=out_ref.at[pl.ds(cw_c * CH, CH), pl.ds(0, HC)],
            send_sem=agcw_send.at[s],
            recv_sem=agcw_recv.at[s],
            device_id=(right,),
            device_id_type=pl.DeviceIdType.MESH,
        )
        ccw = pltpu.make_async_remote_copy(
            src_ref=out_ref.at[pl.ds(ccw_c * CH, CH), pl.ds(HC, HC)],
            dst_ref=out_ref.at[pl.ds(ccw_c * CH, CH), pl.ds(HC, HC)],
            send_sem=agccw_send.at[s],
            recv_sem=agccw_recv.at[s],
            device_id=(left,),
            device_id_type=pl.DeviceIdType.MESH,
        )
        cw.start()
        ccw.start()
        cw.wait()
        ccw.wait()


def _outproj_allreduce(ctx, Wo):
    return pl.pallas_call(
        _ar_body,
        in_specs=[
            pl.BlockSpec(memory_space=pltpu.VMEM),
            pl.BlockSpec(memory_space=pltpu.VMEM),
        ],
        out_specs=pl.BlockSpec(memory_space=pltpu.VMEM),
        out_shape=jax.ShapeDtypeStruct((SQ, DIN), jnp.float32),
        scratch_shapes=[
            pltpu.VMEM((N_DEV - 1, CH, HC), jnp.float32),
            pltpu.VMEM((N_DEV - 1, CH, HC), jnp.float32),
        ] + [pltpu.SemaphoreType.DMA((N_DEV - 1,))] * 8,
        compiler_params=pltpu.CompilerParams(collective_id=0),
    )(ctx, Wo)


def kernel(x, Wq, K_ext, V_ext, Wo):
    i = lax.axis_index("i")
    h0 = i * HPD
    x2 = x.reshape(SQ, DIN)
    K = lax.dynamic_slice(K_ext, (0, 0, h0, 0), (1, SKV, HPD, DH)).reshape(
        SKV, HPD, DH
    ).transpose(1, 0, 2)
    V = lax.dynamic_slice(V_ext, (0, 0, h0, 0), (1, SKV, HPD, DH)).reshape(
        SKV, HPD, DH
    ).transpose(1, 0, 2)
    ctx = _attention(x2, Wq, K, V)
    out = _outproj_allreduce(ctx, Wo)
    return out.reshape(1, SQ, DIN)


# device time: 186245 ns/iter; 1.6149x vs baseline; 1.2520x over previous
import jax
import jax.numpy as jnp
from jax import lax
from jax.experimental import pallas as pl
from jax.experimental.pallas import tpu as pltpu

N_DEV = 4
HPD = 8
SQ = 2048
SKV = 2048
DH = 128
DIN = 1024
DSH = HPD * DH
BQ = 512
SCALE = 0.08838834764831843
WINDOW = 128
NGLOB = 32
CH = SQ // N_DEV


GB = 128
BW = BQ + 2 * WINDOW


def _attn_body(x_ref, wq_ref, k_ref, v_ref, ctx_ref):
    qb = pl.program_id(1)
    q = jnp.dot(x_ref[...], wq_ref[...], preferred_element_type=jnp.float32)

    @pl.when(qb == 0)
    def _dense():
        k = k_ref[0]
        s = lax.dot_general(
            q, k, (((1,), (1,)), ((), ())), preferred_element_type=jnp.float32
        ) * SCALE
        qi = lax.broadcasted_iota(jnp.int32, (BQ, SKV), 0)
        ki = lax.broadcasted_iota(jnp.int32, (BQ, SKV), 1)
        mask = (jnp.abs(qi - ki) <= WINDOW) | (ki < NGLOB) | (qi < NGLOB)
        s = jnp.where(mask, s, -1e9)
        m = jnp.max(s, axis=1, keepdims=True)
        w = jnp.exp(s - m)
        ctx = jnp.dot(w, v_ref[0], preferred_element_type=jnp.float32)
        ctx_ref[...] = ctx / jnp.sum(w, axis=1, keepdims=True)

    @pl.when(qb > 0)
    def _banded():
        bs = jnp.minimum(qb * BQ - WINDOW, SKV - BW)
        off = qb * BQ - bs
        kg = k_ref[0, :GB, :]
        kb = k_ref[0, pl.ds(bs, BW), :]
        s_g = lax.dot_general(
            q, kg, (((1,), (1,)), ((), ())), preferred_element_type=jnp.float32
        ) * SCALE
        s_b = lax.dot_general(
            q, kb, (((1,), (1,)), ((), ())), preferred_element_type=jnp.float32
        ) * SCALE
        kig = lax.broadcasted_iota(jnp.int32, (BQ, GB), 1)
        s_g = jnp.where(kig < NGLOB, s_g, -1e9)
        r = lax.broadcasted_iota(jnp.int32, (BQ, BW), 0)
        j = lax.broadcasted_iota(jnp.int32, (BQ, BW), 1)
        s_b = jnp.where(
            (j >= r + off - WINDOW) & (j <= r + off + WINDOW), s_b, -1e9
        )
        m = jnp.maximum(
            jnp.max(s_g, axis=1, keepdims=True),
            jnp.max(s_b, axis=1, keepdims=True),
        )
        w_g = jnp.exp(s_g - m)
        w_b = jnp.exp(s_b - m)
        denom = (
            jnp.sum(w_g, axis=1, keepdims=True)
            + jnp.sum(w_b, axis=1, keepdims=True)
        )
        ctx = jnp.dot(w_g, v_ref[0, :GB, :], preferred_element_type=jnp.float32)
        ctx += jnp.dot(
            w_b, v_ref[0, pl.ds(bs, BW), :], preferred_element_type=jnp.float32
        )
        ctx_ref[...] = ctx / denom


def _attention(x2, Wq, K, V):
    grid = (HPD, SQ // BQ)
    return pl.pallas_call(
        _attn_body,
        grid=grid,
        in_specs=[
            pl.BlockSpec((BQ, DIN), lambda h, qb: (qb, 0)),
            pl.BlockSpec((DIN, DH), lambda h, qb: (0, h)),
            pl.BlockSpec((1, SKV, DH), lambda h, qb: (h, 0, 0)),
            pl.BlockSpec((1, SKV, DH), lambda h, qb: (h, 0, 0)),
        ],
        out_specs=pl.BlockSpec((BQ, DH), lambda h, qb: (qb, h)),
        out_shape=jax.ShapeDtypeStruct((SQ, DSH), jnp.float32),
        compiler_params=pltpu.CompilerParams(
            dimension_semantics=("arbitrary", "arbitrary"),
        ),
    )(x2, Wq, K, V)


HC = DIN // 2


def _ar_body(ctx_ref, wo_ref, out_ref, cw_comm, ccw_comm,
             cw_send, cw_recv, ccw_send, ccw_recv,
             agcw_send, agcw_recv, agccw_send, agccw_recv):
    i = lax.axis_index("i")
    left = lax.rem(i + N_DEV - 1, N_DEV)
    right = lax.rem(i + 1, N_DEV)

    barrier = pltpu.get_barrier_semaphore()
    for nbr in (left, right):
        pl.semaphore_signal(
            barrier, inc=1, device_id=(nbr,),
            device_id_type=pl.DeviceIdType.MESH,
        )
    pl.semaphore_wait(barrier, 2)

    out_ref[...] = jnp.dot(
        ctx_ref[...], wo_ref[...], preferred_element_type=jnp.float32
    )

    for s in range(N_DEV - 1):
        cw_sc = lax.rem(i - s + N_DEV, N_DEV)
        cw_rc = lax.rem(i - s - 1 + N_DEV, N_DEV)
        ccw_sc = lax.rem(i + s, N_DEV)
        ccw_rc = lax.rem(i + s + 1, N_DEV)
        cw = pltpu.make_async_remote_copy(
            src_ref=out_ref.at[pl.ds(cw_sc * CH, CH), pl.ds(0, HC)],
            dst_ref=cw_comm.at[s],
            send_sem=cw_send.at[s],
            recv_sem=cw_recv.at[s],
            device_id=(right,),
            device_id_type=pl.DeviceIdType.MESH,
        )
        ccw = pltpu.make_async_remote_copy(
            src_ref=out_ref.at[pl.ds(ccw_sc * CH, CH), pl.ds(HC, HC)],
            dst_ref=ccw_comm.at[s],
            send_sem=ccw_send.at[s],
            recv_sem=ccw_recv.at[s],
            device_id=(left,),
            device_id_type=pl.DeviceIdType.MESH,
        )
        cw.start()
        ccw.start()
        cw.wait()
        ccw.wait()
        acc = out_ref[pl.ds(cw_rc * CH, CH), pl.ds(0, HC)] + cw_comm[s]
        out_ref[pl.ds(cw_rc * CH, CH), pl.ds(0, HC)] = acc
        acc = out_ref[pl.ds(ccw_rc * CH, CH), pl.ds(HC, HC)] + ccw_comm[s]
        out_ref[pl.ds(ccw_rc * CH, CH), pl.ds(HC, HC)] = acc

    for s in range(N_DEV - 1):
        cw_c = lax.rem(i + 1 - s + N_DEV, N_DEV)
        ccw_c = lax.rem(i - 1 + s + N_DEV, N_DEV)
        cw = pltpu.make_async_remote_copy(
            src_ref=out_ref.at[pl.ds(cw_c * CH, CH), pl.ds(0, HC)],
            dst_ref=out_ref.at[pl.ds(cw_c * CH, CH), pl.ds(0, HC)],
            send_sem=agcw_send.at[s],
            recv_sem=agcw_recv.at[s],
            device_id=(right,),
            device_id_type=pl.DeviceIdType.MESH,
        )
        ccw = pltpu.make_async_remote_copy(
            src_ref=out_ref.at[pl.ds(ccw_c * CH, CH), pl.ds(HC, HC)],
            dst_ref=out_ref.at[pl.ds(ccw_c * CH, CH), pl.ds(HC, HC)],
            send_sem=agccw_send.at[s],
            recv_sem=agccw_recv.at[s],
            device_id=(left,),
            device_id_type=pl.DeviceIdType.MESH,
        )
        cw.start()
        ccw.start()
        cw.wait()
        ccw.wait()


def _outproj_allreduce(ctx, Wo):
    return pl.pallas_call(
        _ar_body,
        in_specs=[
            pl.BlockSpec(memory_space=pltpu.VMEM),
            pl.BlockSpec(memory_space=pltpu.VMEM),
        ],
        out_specs=pl.BlockSpec(memory_space=pltpu.VMEM),
        out_shape=jax.ShapeDtypeStruct((SQ, DIN), jnp.float32),
        scratch_shapes=[
            pltpu.VMEM((N_DEV - 1, CH, HC), jnp.float32),
            pltpu.VMEM((N_DEV - 1, CH, HC), jnp.float32),
        ] + [pltpu.SemaphoreType.DMA((N_DEV - 1,))] * 8,
        compiler_params=pltpu.CompilerParams(collective_id=0),
    )(ctx, Wo)


def kernel(x, Wq, K_ext, V_ext, Wo):
    i = lax.axis_index("i")
    h0 = i * HPD
    x2 = x.reshape(SQ, DIN)
    K = lax.dynamic_slice(K_ext, (0, 0, h0, 0), (1, SKV, HPD, DH)).reshape(
        SKV, HPD, DH
    ).transpose(1, 0, 2)
    V = lax.dynamic_slice(V_ext, (0, 0, h0, 0), (1, SKV, HPD, DH)).reshape(
        SKV, HPD, DH
    ).transpose(1, 0, 2)
    ctx = _attention(x2, Wq, K, V)
    out = _outproj_allreduce(ctx, Wo)
    return out.reshape(1, SQ, DIN)
